# baseline (speedup 1.0000x reference)
import jax
import jax.numpy as jnp
from jax import lax
from jax.experimental import pallas as pl
from jax.experimental.pallas import tpu as pltpu

B = 16384
D = 64
H = 64
BLK = 1024
UNROLL = 8


def _body(u_idx_ref, i_idx_ref,
          u_tbl_ref, i_tbl_ref,
          w1_ref, b1_ref, w2_ref, b2_ref,
          o_ref,
          u_buf, i_buf, usem, isem):
    base = pl.program_id(0) * BLK

    def issue(g, _):
        r0 = g * UNROLL
        for j in range(UNROLL):
            r = r0 + j
            u = u_idx_ref[base + r]
            pltpu.make_async_copy(u_tbl_ref.at[pl.ds(u, 1)],
                                  u_buf.at[pl.ds(r, 1)], usem).start()
            it = i_idx_ref[base + r]
            pltpu.make_async_copy(i_tbl_ref.at[pl.ds(it, 1)],
                                  i_buf.at[pl.ds(r, 1)], isem).start()
        return 0

    lax.fori_loop(0, BLK // UNROLL, issue, 0)

    def drain(r, _):
        pltpu.make_async_copy(u_tbl_ref.at[pl.ds(0, 1)],
                              u_buf.at[pl.ds(0, 1)], usem).wait()
        pltpu.make_async_copy(i_tbl_ref.at[pl.ds(0, 1)],
                              i_buf.at[pl.ds(0, 1)], isem).wait()
        return 0

    lax.fori_loop(0, BLK, drain, 0)

    w1 = w1_ref[...]
    dn = (((1,), (1,)), ((), ()))
    h = lax.dot_general(u_buf[...], w1[:, :D], dn,
                        preferred_element_type=jnp.float32,
                        precision=lax.Precision.HIGHEST)
    h = h + lax.dot_general(i_buf[...], w1[:, D:], dn,
                            preferred_element_type=jnp.float32,
                            precision=lax.Precision.HIGHEST)
    h = jnp.maximum(h + b1_ref[...], 0.0)
    o = jnp.sum(h * w2_ref[...], axis=1, keepdims=True)
    o_ref[...] = jax.nn.sigmoid(o + b2_ref[0, 0])


def kernel(user_indices, item_indices, user_table, item_table, W1, b1, W2, b2):
    grid_spec = pltpu.PrefetchScalarGridSpec(
        num_scalar_prefetch=2,
        grid=(B // BLK,),
        in_specs=[
            pl.BlockSpec(memory_space=pltpu.MemorySpace.HBM),
            pl.BlockSpec(memory_space=pltpu.MemorySpace.HBM),
            pl.BlockSpec((H, 2 * D), lambda i, *_: (0, 0)),
            pl.BlockSpec((1, H), lambda i, *_: (0, 0)),
            pl.BlockSpec((1, H), lambda i, *_: (0, 0)),
            pl.BlockSpec((1, 1), lambda i, *_: (0, 0)),
        ],
        out_specs=pl.BlockSpec((BLK, 1), lambda i, *_: (i, 0)),
        scratch_shapes=[
            pltpu.VMEM((BLK, D), jnp.float32),
            pltpu.VMEM((BLK, D), jnp.float32),
            pltpu.SemaphoreType.DMA,
            pltpu.SemaphoreType.DMA,
        ],
    )
    out = pl.pallas_call(
        _body,
        grid_spec=grid_spec,
        out_shape=jax.ShapeDtypeStruct((B, 1), jnp.float32),
    )(user_indices.astype(jnp.int32), item_indices.astype(jnp.int32),
      user_table, item_table,
      W1, b1.reshape(1, H), W2, b2.reshape(1, 1))
    return out.reshape(B)


# speedup vs baseline: 1.0964x; 1.0964x over previous
import jax
import jax.numpy as jnp
from jax import lax
from jax.experimental import pallas as pl
from jax.experimental.pallas import tpu as pltpu

B = 16384
D = 64
H = 64
BLK = 1024
UNROLL = 8


def _body(u_idx_ref, i_idx_ref,
          u_tbl_ref, i_tbl_ref,
          w1_ref, b1_ref, w2_ref, b2_ref,
          o_ref,
          u_buf, i_buf, usem, isem):
    base = pl.program_id(0) * BLK

    def issue(g, _):
        r0 = g * UNROLL
        for j in range(UNROLL):
            r = r0 + j
            u = u_idx_ref[base + r]
            pltpu.make_async_copy(u_tbl_ref.at[pl.ds(u, 1)],
                                  u_buf.at[pl.ds(r, 1)], usem).start()
            it = i_idx_ref[base + r]
            pltpu.make_async_copy(i_tbl_ref.at[pl.ds(it, 1)],
                                  i_buf.at[pl.ds(r, 1)], isem).start()
        return 0

    lax.fori_loop(0, BLK // UNROLL, issue, 0)

    pltpu.make_async_copy(u_tbl_ref.at[pl.ds(0, BLK)], u_buf, usem).wait()
    pltpu.make_async_copy(i_tbl_ref.at[pl.ds(0, BLK)], i_buf, isem).wait()

    w1 = w1_ref[...]
    dn = (((1,), (1,)), ((), ()))
    h = lax.dot_general(u_buf[...], w1[:, :D], dn,
                        preferred_element_type=jnp.float32,
                        precision=lax.Precision.HIGHEST)
    h = h + lax.dot_general(i_buf[...], w1[:, D:], dn,
                            preferred_element_type=jnp.float32,
                            precision=lax.Precision.HIGHEST)
    h = jnp.maximum(h + b1_ref[...], 0.0)
    o = jnp.sum(h * w2_ref[...], axis=1, keepdims=True)
    o_ref[...] = jax.nn.sigmoid(o + b2_ref[0, 0])


def kernel(user_indices, item_indices, user_table, item_table, W1, b1, W2, b2):
    grid_spec = pltpu.PrefetchScalarGridSpec(
        num_scalar_prefetch=2,
        grid=(B // BLK,),
        in_specs=[
            pl.BlockSpec(memory_space=pltpu.MemorySpace.HBM),
            pl.BlockSpec(memory_space=pltpu.MemorySpace.HBM),
            pl.BlockSpec((H, 2 * D), lambda i, *_: (0, 0)),
            pl.BlockSpec((1, H), lambda i, *_: (0, 0)),
            pl.BlockSpec((1, H), lambda i, *_: (0, 0)),
            pl.BlockSpec((1, 1), lambda i, *_: (0, 0)),
        ],
        out_specs=pl.BlockSpec((BLK, 1), lambda i, *_: (i, 0)),
        scratch_shapes=[
            pltpu.VMEM((BLK, D), jnp.float32),
            pltpu.VMEM((BLK, D), jnp.float32),
            pltpu.SemaphoreType.DMA,
            pltpu.SemaphoreType.DMA,
        ],
    )
    out = pl.pallas_call(
        _body,
        grid_spec=grid_spec,
        out_shape=jax.ShapeDtypeStruct((B, 1), jnp.float32),
    )(user_indices.astype(jnp.int32), item_indices.astype(jnp.int32),
      user_table, item_table,
      W1, b1.reshape(1, H), W2, b2.reshape(1, 1))
    return out.reshape(B)
